# baseline (speedup 1.0000x reference)
import functools

import jax
import jax.numpy as jnp
from jax import lax
from jax.experimental import pallas as pl
from jax.experimental.pallas import tpu as pltpu
from jax.experimental.pallas import tpu_sc as plsc

B, T, MOD, DEPTH, WORD, MEM = 8, 10, 3, 3, 256, 256
NV = T * MOD * DEPTH
NROW = 93


def _tc_body(x_any, we_any, be_ref, wa_any, ba_ref, demo_ref, w1_any, b1_ref,
             w2_any, b2_ref, wf_any, bf_ref, mask_ref, lab90_ref, root_ref,
             init_ref, p_hbm, q_hbm, fir_hbm, img_hbm,
             x_s, we_s, wa_s, w1_s, w2_s, wf_s,
             p_s, q_s, fir_s, img_s, sem, sem2):
    f32 = jnp.float32
    dn = (((1,), (1,)), ((), ()))

    ins = [pltpu.make_async_copy(src, dst, sem2) for src, dst in
           ((x_any, x_s), (we_any, we_s), (wa_any, wa_s), (w1_any, w1_s),
            (w2_any, w2_s), (wf_any, wf_s))]
    for d in ins:
        d.start()
    for d in ins:
        d.wait()
    x = x_s[...]
    we_ref, wa_ref, w1_ref, w2_ref, wf_ref = we_s, wa_s, w1_s, w2_s, wf_s
    E = jax.nn.sigmoid(
        lax.dot_general(x, we_ref[...], dn, preferred_element_type=f32)
        + be_ref[...])
    A = jnp.tanh(
        lax.dot_general(x, wa_ref[...], dn, preferred_element_type=f32)
        + ba_ref[...])

    Eg = E.reshape(B * T * MOD, DEPTH, WORD)
    Ag = A.reshape(B * T * MOD, DEPTH, WORD)
    u1h = 1.0 - 0.5 * Eg[:, 1, :]
    u2h = 1.0 - 0.5 * Eg[:, 2, :]
    u2q = 1.0 - 0.25 * Eg[:, 2, :]
    a0, a1, a2 = Ag[:, 0, :], Ag[:, 1, :], Ag[:, 2, :]
    P2 = 1.0 - Eg[:, 2, :]
    Q2 = a2
    P1 = (1.0 - Eg[:, 1, :]) * u2h
    Q1 = a1 * u2h + 0.5 * a2
    P0 = (1.0 - Eg[:, 0, :]) * u1h * u2q
    Q0 = a0 * u1h * u2q + 0.5 * a1 * u2q + 0.25 * a2
    p_s[...] = jnp.stack([P0, P1, P2], axis=1).reshape(B, NV, WORD)
    q_s[...] = jnp.stack([Q0, Q1, Q2], axis=1).reshape(B, NV, WORD)
    d1 = pltpu.make_async_copy(p_s, p_hbm, sem)
    d2 = pltpu.make_async_copy(q_s, q_hbm, sem)
    d1.start()
    d2.start()

    demo = demo_ref[...]
    h = jax.nn.relu(
        lax.dot_general(demo, w1_ref[...], dn, preferred_element_type=f32)
        + b1_ref[...])
    h = (lax.dot_general(h, w2_ref[...], dn, preferred_element_type=f32)
         + b2_ref[...] + demo)
    de = (lax.dot_general(h, wf_ref[...], dn, preferred_element_type=f32)
          + bf_ref[...])

    mi = lax.broadcasted_iota(jnp.int32, (NV, NV), 0)
    ni = lax.broadcasted_iota(jnp.int32, (NV, NV), 1)
    ii, li = mi % 3, ni % 3
    pw = jnp.where(ii == 0, 1.0, jnp.where(ii == 1, 51.0, 51.0 * 51.0))
    sel = jnp.where((mi // 3 == ni // 3) & (ii <= li), pw, 0.0)
    li2 = lax.broadcasted_iota(jnp.int32, (B, NV), 1) % 3
    bias = jnp.where(li2 == 0, 1.0,
                     jnp.where(li2 == 1, 52.0, 52.0 + 51.0 * 51.0))
    dn2 = (((1,), (0,)), ((), ()))
    codes = (lax.dot_general(lab90_ref[...], sel, dn2,
                             preferred_element_type=f32) + bias)
    rep = jnp.where(lax.broadcasted_iota(jnp.int32, (T, NV), 1) // 9
                    == lax.broadcasted_iota(jnp.int32, (T, NV), 0), 1.0, 0.0)
    mskf = jnp.where(mask_ref[...] != 0, 1.0, 0.0)
    act = lax.dot_general(mskf, rep, dn2, preferred_element_type=f32) > 0.5

    m3 = lax.broadcasted_iota(jnp.int32, (B, NV, NV), 2)
    eq = (codes[:, :, None] == codes[:, None, :]) & act[:, None, :]
    first = jnp.min(jnp.where(eq, m3, NV), axis=2)
    n2 = lax.broadcasted_iota(jnp.int32, (B, NV), 1)
    is_first = act & (first == n2)
    count = jnp.sum(jnp.where(is_first, 1, 0), axis=1)
    idx = jnp.sum(
        jnp.where(is_first[:, None, :] & (m3 <= first[:, :, None]), 1, 0),
        axis=2) - 1
    row = jnp.where(act, idx + 2, NROW - 1)
    fir_s[...] = jnp.broadcast_to(row[:, :, None], (B, NV, 16))
    d3 = pltpu.make_async_copy(fir_s, fir_hbm, sem)
    d3.start()

    g = jnp.where(n2 < count[:, None], 1.0, 0.0).astype(f32)
    root2 = jnp.broadcast_to(root_ref[...][None, None, :], (B, 1, MEM))
    slots0 = g[:, :, None] * init_ref[...][None, None, :]
    img_s[...] = jnp.concatenate([root2, de[:, None, :], slots0], axis=1)
    d4 = pltpu.make_async_copy(img_s, img_hbm, sem)
    d4.start()
    d1.wait()
    d2.wait()
    d3.wait()
    d4.wait()


def _sc_body(p_hbm, q_hbm, fir_hbm, init_hbm, out_hbm, p_v, q_v, fir_v, stage,
             sem):
    info = plsc.get_sparse_core_info()
    nc = info.num_cores
    wid = lax.axis_index("s") * nc + lax.axis_index("c")
    b = wid // 2
    off = (wid % 2) * 128

    @pl.when(wid < 16)
    def _():
        d1 = pltpu.async_copy(p_hbm.at[b, :, pl.ds(off, 128)], p_v, sem)
        d2 = pltpu.async_copy(q_hbm.at[b, :, pl.ds(off, 128)], q_v, sem)
        d3 = pltpu.async_copy(fir_hbm.at[b], fir_v, sem)
        d4 = pltpu.async_copy(init_hbm.at[b, :, pl.ds(off, 128)],
                              stage.at[pl.ds(0, 92)], sem)
        d1.wait(); d2.wait(); d3.wait(); d4.wait()

        iota = lax.iota(jnp.int32, 16)
        zeros = jnp.zeros((16,), jnp.float32)
        for c in range(8):
            stage[NROW - 1, pl.ds(c * 16, 16)] = zeros

        def step(n, carry):
            rowv = fir_v[n, :]

            @plsc.parallel_loop(0, 128, step=16, unroll=8)
            def _inner(c16):
                col = iota + c16
                sl = pl.ds(c16, 16)
                cur = plsc.load_gather(stage, [rowv, col])
                plsc.store_scatter(stage, [rowv, col],
                                   cur * p_v[n, sl] + q_v[n, sl])

            return carry

        lax.fori_loop(0, NV, step, 0)
        pltpu.sync_copy(stage.at[pl.ds(0, 92)], out_hbm.at[b, :, pl.ds(off, 128)])


@functools.lru_cache(maxsize=1)
def _make_sc_compose():
    mesh = plsc.VectorSubcoreMesh(core_axis_name="c", subcore_axis_name="s")
    return pl.kernel(
        _sc_body,
        out_type=jax.ShapeDtypeStruct((B, 92, MEM), jnp.float32),
        mesh=mesh,
        compiler_params=pltpu.CompilerParams(needs_layout_passes=False,
                                             skip_device_barrier=True),
        scratch_types=[
            pltpu.VMEM((NV, 128), jnp.float32),
            pltpu.VMEM((NV, 128), jnp.float32),
            pltpu.VMEM((NV, 16), jnp.int32),
            pltpu.VMEM((NROW, 128), jnp.float32),
            pltpu.SemaphoreType.DMA,
        ],
    )


def kernel(input, mask, labels, demo, W1, b1, W2, b2, Wf, bf, We, be, Wa, ba,
           init_mem, root_mem):
    P, Q, fir, img = pl.pallas_call(
        _tc_body,
        out_shape=[
            jax.ShapeDtypeStruct((B, NV, WORD), jnp.float32),
            jax.ShapeDtypeStruct((B, NV, WORD), jnp.float32),
            jax.ShapeDtypeStruct((B, NV, 16), jnp.int32),
            jax.ShapeDtypeStruct((B, 92, MEM), jnp.float32),
        ],
        in_specs=[
            pl.BlockSpec(memory_space=pltpu.HBM),
            pl.BlockSpec(memory_space=pltpu.HBM),
            pl.BlockSpec(memory_space=pltpu.VMEM),
            pl.BlockSpec(memory_space=pltpu.HBM),
            pl.BlockSpec(memory_space=pltpu.VMEM),
            pl.BlockSpec(memory_space=pltpu.VMEM),
            pl.BlockSpec(memory_space=pltpu.HBM),
            pl.BlockSpec(memory_space=pltpu.VMEM),
            pl.BlockSpec(memory_space=pltpu.HBM),
            pl.BlockSpec(memory_space=pltpu.VMEM),
            pl.BlockSpec(memory_space=pltpu.HBM),
            pl.BlockSpec(memory_space=pltpu.VMEM),
            pl.BlockSpec(memory_space=pltpu.VMEM),
            pl.BlockSpec(memory_space=pltpu.VMEM),
            pl.BlockSpec(memory_space=pltpu.VMEM),
            pl.BlockSpec(memory_space=pltpu.VMEM),
        ],
        out_specs=[pl.BlockSpec(memory_space=pltpu.HBM)] * 4,
        scratch_shapes=[
            pltpu.VMEM((B * NV, WORD), jnp.float32),
            pltpu.VMEM((MEM, WORD), jnp.float32),
            pltpu.VMEM((MEM, WORD), jnp.float32),
            pltpu.VMEM((512, 64), jnp.float32),
            pltpu.VMEM((64, 512), jnp.float32),
            pltpu.VMEM((MEM, 64), jnp.float32),
            pltpu.VMEM((B, NV, WORD), jnp.float32),
            pltpu.VMEM((B, NV, WORD), jnp.float32),
            pltpu.VMEM((B, NV, 16), jnp.int32),
            pltpu.VMEM((B, 92, MEM), jnp.float32),
            pltpu.SemaphoreType.DMA,
            pltpu.SemaphoreType.DMA,
        ],
    )(input.reshape(B * NV, WORD), We, be, Wa, ba, demo, W1, b1, W2, b2,
      Wf, bf, mask, labels.reshape(B, NV).astype(jnp.float32),
      root_mem, init_mem)

    return _make_sc_compose()(P, Q, fir, img)


# speedup vs baseline: 1.0447x; 1.0447x over previous
import functools

import jax
import jax.numpy as jnp
from jax import lax
from jax.experimental import pallas as pl
from jax.experimental.pallas import tpu as pltpu
from jax.experimental.pallas import tpu_sc as plsc

B, T, MOD, DEPTH, WORD, MEM = 8, 10, 3, 3, 256, 256
NV = T * MOD * DEPTH
NROW = 93


def _tc_body(x_ref, we_ref, be_ref, wa_ref, ba_ref, demo_ref, w1_ref, b1_ref,
             w2_ref, b2_ref, wf_ref, bf_ref, mask_ref, lab90_ref, root_ref,
             init_ref, p_hbm, q_hbm, fir_hbm, img_hbm,
             p_s, q_s, fir_s, img_s, sem):
    f32 = jnp.float32
    dn = (((1,), (1,)), ((), ()))

    x = x_ref[...]
    E = jax.nn.sigmoid(
        lax.dot_general(x, we_ref[...], dn, preferred_element_type=f32)
        + be_ref[...])
    A = jnp.tanh(
        lax.dot_general(x, wa_ref[...], dn, preferred_element_type=f32)
        + ba_ref[...])

    Eg = E.reshape(B * T * MOD, DEPTH, WORD)
    Ag = A.reshape(B * T * MOD, DEPTH, WORD)
    u1h = 1.0 - 0.5 * Eg[:, 1, :]
    u2h = 1.0 - 0.5 * Eg[:, 2, :]
    u2q = 1.0 - 0.25 * Eg[:, 2, :]
    a0, a1, a2 = Ag[:, 0, :], Ag[:, 1, :], Ag[:, 2, :]
    P2 = 1.0 - Eg[:, 2, :]
    Q2 = a2
    P1 = (1.0 - Eg[:, 1, :]) * u2h
    Q1 = a1 * u2h + 0.5 * a2
    P0 = (1.0 - Eg[:, 0, :]) * u1h * u2q
    Q0 = a0 * u1h * u2q + 0.5 * a1 * u2q + 0.25 * a2
    G = T * MOD
    p_s[...] = jnp.stack(
        [P0.reshape(B, G, WORD), P1.reshape(B, G, WORD),
         P2.reshape(B, G, WORD)], axis=1).reshape(B, NV, WORD)
    q_s[...] = jnp.stack(
        [Q0.reshape(B, G, WORD), Q1.reshape(B, G, WORD),
         Q2.reshape(B, G, WORD)], axis=1).reshape(B, NV, WORD)
    d1 = pltpu.make_async_copy(p_s, p_hbm, sem)
    d2 = pltpu.make_async_copy(q_s, q_hbm, sem)
    d1.start()
    d2.start()

    demo = demo_ref[...]
    h = jax.nn.relu(
        lax.dot_general(demo, w1_ref[...], dn, preferred_element_type=f32)
        + b1_ref[...])
    h = (lax.dot_general(h, w2_ref[...], dn, preferred_element_type=f32)
         + b2_ref[...] + demo)
    de = (lax.dot_general(h, wf_ref[...], dn, preferred_element_type=f32)
          + bf_ref[...])

    mi = lax.broadcasted_iota(jnp.int32, (NV, NV), 0)
    ni = lax.broadcasted_iota(jnp.int32, (NV, NV), 1)
    ii, li = mi % 3, ni % 3
    pw = jnp.where(ii == 0, 1.0, jnp.where(ii == 1, 51.0, 51.0 * 51.0))
    sel = jnp.where((mi // 3 == ni // 3) & (ii <= li), pw, 0.0)
    li2 = lax.broadcasted_iota(jnp.int32, (B, NV), 1) % 3
    bias = jnp.where(li2 == 0, 1.0,
                     jnp.where(li2 == 1, 52.0, 52.0 + 51.0 * 51.0))
    dn2 = (((1,), (0,)), ((), ()))
    codes = (lax.dot_general(lab90_ref[...], sel, dn2,
                             preferred_element_type=f32) + bias)
    rep = jnp.where(lax.broadcasted_iota(jnp.int32, (T, NV), 1) // 9
                    == lax.broadcasted_iota(jnp.int32, (T, NV), 0), 1.0, 0.0)
    mskf = jnp.where(mask_ref[...] != 0, 1.0, 0.0)
    act = lax.dot_general(mskf, rep, dn2, preferred_element_type=f32) > 0.5

    m3 = lax.broadcasted_iota(jnp.int32, (B, NV, NV), 2)
    eq = (codes[:, :, None] == codes[:, None, :]) & act[:, None, :]
    first = jnp.min(jnp.where(eq, m3, NV), axis=2)
    n2 = lax.broadcasted_iota(jnp.int32, (B, NV), 1)
    is_first = act & (first == n2)
    count = jnp.sum(jnp.where(is_first, 1, 0), axis=1)
    idx = jnp.sum(
        jnp.where(is_first[:, None, :] & (m3 <= first[:, :, None]), 1, 0),
        axis=2) - 1
    row = jnp.where(act, idx + 2, NROW - 1)
    fir_s[...] = jnp.broadcast_to(row[:, :, None], (B, NV, 16))
    d3 = pltpu.make_async_copy(fir_s, fir_hbm, sem)
    d3.start()

    g = jnp.where(n2 < count[:, None], 1.0, 0.0).astype(f32)
    root2 = jnp.broadcast_to(root_ref[...][None, None, :], (B, 1, MEM))
    slots0 = g[:, :, None] * init_ref[...][None, None, :]
    img_s[...] = jnp.concatenate([root2, de[:, None, :], slots0], axis=1)
    d4 = pltpu.make_async_copy(img_s, img_hbm, sem)
    d4.start()
    d1.wait()
    d2.wait()
    d3.wait()
    d4.wait()


def _sc_body(p_hbm, q_hbm, fir_hbm, init_hbm, out_hbm, p_v, q_v, fir_v, stage,
             sem):
    info = plsc.get_sparse_core_info()
    nc = info.num_cores
    wid = lax.axis_index("s") * nc + lax.axis_index("c")
    b = wid // 2
    off = (wid % 2) * 128

    @pl.when(wid < 16)
    def _():
        d1 = pltpu.async_copy(p_hbm.at[b, :, pl.ds(off, 128)], p_v, sem)
        d2 = pltpu.async_copy(q_hbm.at[b, :, pl.ds(off, 128)], q_v, sem)
        d3 = pltpu.async_copy(fir_hbm.at[b], fir_v, sem)
        d4 = pltpu.async_copy(init_hbm.at[b, :, pl.ds(off, 128)],
                              stage.at[pl.ds(0, 92)], sem)
        d1.wait(); d2.wait(); d3.wait(); d4.wait()

        iota = lax.iota(jnp.int32, 16)
        zeros = jnp.zeros((16,), jnp.float32)
        for c in range(8):
            stage[NROW - 1, pl.ds(c * 16, 16)] = zeros

        def step(n, carry):
            rowv = fir_v[n, :]
            pn = (n % 3) * (T * MOD) + n // 3

            @plsc.parallel_loop(0, 128, step=16, unroll=8)
            def _inner(c16):
                col = iota + c16
                sl = pl.ds(c16, 16)
                cur = plsc.load_gather(stage, [rowv, col])
                plsc.store_scatter(stage, [rowv, col],
                                   cur * p_v[pn, sl] + q_v[pn, sl])

            return carry

        lax.fori_loop(0, NV, step, 0, unroll=2)
        pltpu.sync_copy(stage.at[pl.ds(0, 92)], out_hbm.at[b, :, pl.ds(off, 128)])


@functools.lru_cache(maxsize=1)
def _make_sc_compose():
    mesh = plsc.VectorSubcoreMesh(core_axis_name="c", subcore_axis_name="s")
    return pl.kernel(
        _sc_body,
        out_type=jax.ShapeDtypeStruct((B, 92, MEM), jnp.float32),
        mesh=mesh,
        compiler_params=pltpu.CompilerParams(needs_layout_passes=False,
                                             skip_device_barrier=True),
        scratch_types=[
            pltpu.VMEM((NV, 128), jnp.float32),
            pltpu.VMEM((NV, 128), jnp.float32),
            pltpu.VMEM((NV, 16), jnp.int32),
            pltpu.VMEM((NROW, 128), jnp.float32),
            pltpu.SemaphoreType.DMA,
        ],
    )


def kernel(input, mask, labels, demo, W1, b1, W2, b2, Wf, bf, We, be, Wa, ba,
           init_mem, root_mem):
    P, Q, fir, img = pl.pallas_call(
        _tc_body,
        out_shape=[
            jax.ShapeDtypeStruct((B, NV, WORD), jnp.float32),
            jax.ShapeDtypeStruct((B, NV, WORD), jnp.float32),
            jax.ShapeDtypeStruct((B, NV, 16), jnp.int32),
            jax.ShapeDtypeStruct((B, 92, MEM), jnp.float32),
        ],
        out_specs=[pl.BlockSpec(memory_space=pltpu.HBM)] * 4,
        scratch_shapes=[
            pltpu.VMEM((B, NV, WORD), jnp.float32),
            pltpu.VMEM((B, NV, WORD), jnp.float32),
            pltpu.VMEM((B, NV, 16), jnp.int32),
            pltpu.VMEM((B, 92, MEM), jnp.float32),
            pltpu.SemaphoreType.DMA,
        ],
    )(input.reshape(B * NV, WORD), We, be, Wa, ba, demo, W1, b1, W2, b2,
      Wf, bf, mask, labels.reshape(B, NV).astype(jnp.float32),
      root_mem, init_mem)

    return _make_sc_compose()(P, Q, fir, img)
